# baseline (speedup 1.0000x reference)
import functools

import jax
import jax.numpy as jnp
from jax import lax
from jax.experimental import pallas as pl
from jax.experimental.pallas import tpu as pltpu
from jax.experimental.pallas import tpu_sc as plsc

B = 4096 * 200
D = 64
NC = 2
NS = 16
NW = NC * NS
BPW = B // NW
K = 128
NBUF = 4
NGROUP = BPW // (K * NBUF)

_mesh = plsc.VectorSubcoreMesh(core_axis_name="c", subcore_axis_name="s")


@functools.partial(
    pl.kernel,
    mesh=_mesh,
    out_type=jax.ShapeDtypeStruct((B, D), jnp.float32),
    scratch_types=[
        pltpu.VMEM((BPW,), jnp.int32),
        pltpu.VMEM((NBUF, K, D), jnp.float32),
        pltpu.SemaphoreType.DMA,
        pltpu.SemaphoreType.DMA,
    ],
    compiler_params=pltpu.CompilerParams(use_tc_tiling_on_sc=False),
)
def _emb_gather(idx_hbm, table_hbm, out_hbm, idx_v, rows_v, gsem, osem):
    wid = lax.axis_index("s") * NC + lax.axis_index("c")
    base = wid * BPW
    pltpu.sync_copy(idx_hbm.at[pl.ds(base, BPW)], idx_v)

    def group(g, carry):
        first = g * (K * NBUF)
        for b in range(NBUF):
            pltpu.make_async_copy(
                table_hbm.at[idx_v.at[pl.ds(first + b * K, K)]],
                rows_v.at[b], gsem).start()
        for b in range(NBUF):
            pltpu.make_async_copy(
                table_hbm.at[idx_v.at[pl.ds(first + b * K, K)]],
                rows_v.at[b], gsem).wait()
        for b in range(NBUF):
            pltpu.make_async_copy(
                rows_v.at[b],
                out_hbm.at[pl.ds(base + first + b * K, K)], osem).start()
        for b in range(NBUF):
            pltpu.make_async_copy(
                rows_v.at[b],
                out_hbm.at[pl.ds(base + first + b * K, K)], osem).wait()
        return carry

    lax.fori_loop(0, NGROUP, group, None)


def kernel(idxs, weight):
    flat = idxs.reshape(-1).astype(jnp.int32)
    out = _emb_gather(flat, weight)
    return out.reshape(idxs.shape + (weight.shape[-1],))


# speedup vs baseline: 1.0259x; 1.0259x over previous
import functools

import jax
import jax.numpy as jnp
from jax import lax
from jax.experimental import pallas as pl
from jax.experimental.pallas import tpu as pltpu
from jax.experimental.pallas import tpu_sc as plsc

B = 4096 * 200
D = 64
NC = 2
NS = 16
NW = NC * NS
BPW = B // NW
K = 128
NBUF = 4
NSET = BPW // (K * NBUF)

_mesh = plsc.VectorSubcoreMesh(core_axis_name="c", subcore_axis_name="s")


@functools.partial(
    pl.kernel,
    mesh=_mesh,
    out_type=jax.ShapeDtypeStruct((B, D), jnp.float32),
    scratch_types=[
        pltpu.VMEM((BPW,), jnp.int32),
        pltpu.VMEM((2, NBUF, K, D), jnp.float32),
        pltpu.SemaphoreType.DMA,
        pltpu.SemaphoreType.DMA,
    ],
    compiler_params=pltpu.CompilerParams(use_tc_tiling_on_sc=False),
)
def _emb_gather(idx_hbm, table_hbm, out_hbm, idx_v, rows_v, gsem, osem):
    wid = lax.axis_index("s") * NC + lax.axis_index("c")
    base = wid * BPW
    pltpu.sync_copy(idx_hbm.at[pl.ds(base, BPW)], idx_v)

    def fire_gathers(s, half):
        for b in range(NBUF):
            pltpu.make_async_copy(
                table_hbm.at[idx_v.at[pl.ds(s * (K * NBUF) + b * K, K)]],
                rows_v.at[half, b], gsem).start()

    def drain_gathers(half):
        for b in range(NBUF):
            pltpu.make_async_copy(
                table_hbm.at[idx_v.at[pl.ds(b * K, K)]],
                rows_v.at[half, b], gsem).wait()

    def fire_writes(s, half):
        for b in range(NBUF):
            pltpu.make_async_copy(
                rows_v.at[half, b],
                out_hbm.at[pl.ds(base + s * (K * NBUF) + b * K, K)],
                osem).start()

    def drain_writes(half):
        for b in range(NBUF):
            pltpu.make_async_copy(
                rows_v.at[half, b],
                out_hbm.at[pl.ds(base, K)], osem).wait()

    fire_gathers(0, 0)

    def step2(i, carry):
        g = i * 2
        for p in range(2):
            s = g + p
            cur, other = p, 1 - p

            @pl.when(s > 0)
            def _():
                drain_writes(other)

            @pl.when(s + 1 < NSET)
            def _():
                fire_gathers(s + 1, other)

            drain_gathers(cur)
            fire_writes(s, cur)
        return carry

    lax.fori_loop(0, NSET // 2, step2, None)
    drain_writes(1)


def kernel(idxs, weight):
    flat = idxs.reshape(-1).astype(jnp.int32)
    out = _emb_gather(flat, weight)
    return out.reshape(idxs.shape + (weight.shape[-1],))
